# baseline (speedup 1.0000x reference)
import functools

import jax
import jax.numpy as jnp
from jax import lax
from jax.experimental import pallas as pl
from jax.experimental.pallas import tpu as pltpu
from jax.experimental.pallas import tpu_sc as plsc

_B, _S, _D = 4, 4096, 768
_N = _B * _S
_NW = 32
_P = _S // _NW
_C = 32
_NCHUNK = _P // _C
_NVREG = _D // 16


def _emb_kernel(x_flat, tok_table, pe):
    mesh = plsc.VectorSubcoreMesh(core_axis_name="c", subcore_axis_name="s")

    @functools.partial(
        pl.kernel,
        out_type=jax.ShapeDtypeStruct((_N, _D), jnp.float32),
        mesh=mesh,
        scratch_types=[
            pltpu.VMEM((_P, _D), jnp.float32),
            pltpu.VMEM((_C, _D), jnp.float32),
            pltpu.VMEM((_C,), jnp.int32),
        ],
    )
    def body(x_hbm, table_hbm, pe_hbm, out_hbm, pe_v, buf, idx_v):
        wid = lax.axis_index("s") * 2 + lax.axis_index("c")
        s0 = wid * _P
        pltpu.sync_copy(pe_hbm.at[pl.ds(s0, _P)], pe_v)
        for b in range(_B):
            for c in range(_NCHUNK):
                p0 = c * _C
                r0 = b * _S + s0 + p0
                pltpu.sync_copy(x_hbm.at[pl.ds(r0, _C)], idx_v)
                pltpu.sync_copy(table_hbm.at[idx_v], buf)

                @pl.loop(0, _C)
                def _(r):
                    for k in range(_NVREG):
                        v = pe_v[p0 + r, pl.ds(k * 16, 16)]
                        plsc.addupdate(buf.at[r, pl.ds(k * 16, 16)], v)

                pltpu.sync_copy(buf, out_hbm.at[pl.ds(r0, _C)])

    return body(x_flat, tok_table, pe)


def kernel(x, tok_table, pe):
    out = _emb_kernel(x.reshape(_N), tok_table, pe)
    return out.reshape(_B, _S, _D)


# speedup vs baseline: 1.9722x; 1.9722x over previous
import functools

import jax
import jax.numpy as jnp
from jax import lax
from jax.experimental import pallas as pl
from jax.experimental.pallas import tpu as pltpu
from jax.experimental.pallas import tpu_sc as plsc

_B, _S, _D = 4, 4096, 768
_N = _B * _S
_NW = 32
_P = _S // _NW
_HALF = _P // 2
_C = 32
_NVREG = _D // 16
_NCHUNK = _B * _P // _C
_NBUF = 3


def _emb_kernel(x_flat, tok_table, pe):
    mesh = plsc.VectorSubcoreMesh(core_axis_name="c", subcore_axis_name="s")

    @functools.partial(
        pl.kernel,
        out_type=jax.ShapeDtypeStruct((_N, _D), jnp.float32),
        mesh=mesh,
        scratch_types=[
            pltpu.VMEM((_HALF, _D), jnp.float32),
            [pltpu.VMEM((_C, _D), jnp.float32) for _ in range(_NBUF)],
            pltpu.VMEM((_B * _P,), jnp.int32),
            [pltpu.SemaphoreType.DMA for _ in range(_NBUF)],
            [pltpu.SemaphoreType.DMA for _ in range(_NBUF)],
            pltpu.SemaphoreType.DMA,
        ],
    )
    def body(x_hbm, table_hbm, pe_hbm, out_hbm, pe_v, bufs, idx_v,
             gsems, wsems, pe_sem):
        wid = lax.axis_index("s") * 2 + lax.axis_index("c")
        s0 = wid * _P

        def chunk_coords(k):
            h, b, c = k // 8, (k % 8) // 2, k % 2
            poff = h * _HALF + c * _C
            return h, b, c, poff

        def start_gather(k):
            _, b, c, poff = chunk_coords(k)
            ioff = b * _P + poff
            return pltpu.async_copy(
                table_hbm.at[idx_v.at[pl.ds(ioff, _C)]],
                bufs[k % _NBUF], gsems[k % _NBUF])

        def start_pe_load(h):
            return pltpu.async_copy(
                pe_hbm.at[pl.ds(s0 + h * _HALF, _HALF)], pe_v, pe_sem)

        for b in range(_B):
            pltpu.sync_copy(x_hbm.at[pl.ds(b * _S + s0, _P)],
                            idx_v.at[pl.ds(b * _P, _P)])
        pe_d = start_pe_load(0)
        gd = {0: start_gather(0), 1: start_gather(1)}
        wd = {}

        for k in range(_NCHUNK):
            h, b, c, poff = chunk_coords(k)
            if k == 0 or k == 8:
                pe_d.wait()
            gd.pop(k).wait()
            buf = bufs[k % _NBUF]
            prow = c * _C

            @plsc.parallel_loop(0, _C)
            def _(r):
                for j in range(_NVREG):
                    v = pe_v[prow + r, pl.ds(j * 16, 16)]
                    plsc.addupdate(buf.at[r, pl.ds(j * 16, 16)], v)

            if k == 7:
                pe_d = start_pe_load(1)
            row0 = b * _S + s0 + poff
            wd[k] = pltpu.async_copy(buf, out_hbm.at[pl.ds(row0, _C)],
                                     wsems[k % _NBUF])
            nk = k + _NBUF - 1
            if nk < _NCHUNK and nk - _NBUF >= 0:
                wd.pop(nk - _NBUF).wait()
            if nk < _NCHUNK:
                gd[nk] = start_gather(nk)
        for d in wd.values():
            d.wait()

    return body(x_flat, tok_table, pe)


def kernel(x, tok_table, pe):
    out = _emb_kernel(x.reshape(_N), tok_table, pe)
    return out.reshape(_B, _S, _D)
